# baseline (speedup 1.0000x reference)
import functools

import jax
import jax.numpy as jnp
from jax import lax
from jax.experimental import pallas as pl
from jax.experimental.pallas import tpu as pltpu
from jax.experimental.pallas import tpu_sc as plsc

_NC = 2
_NS = 16
_NW = _NC * _NS
_LANES = 16
_CH = 64
_BAT = 4
_BN = 1000
_BE = 2048


def _full_spec(shape):
    nd = len(shape)
    return pl.BlockSpec(shape, lambda i, _n=nd: (0,) * _n)


def _xs_body(x_ref, w_ref, o_ref):
    o_ref[...] = jnp.dot(x_ref[...], w_ref[...],
                         preferred_element_type=jnp.float32)


def _phi_body(rbf_t_ref, w1_ref, b1_ref, w2_ref, b2_ref, o_ref):
    h = lax.dot_general(rbf_t_ref[...], w1_ref[...],
                        (((0,), (0,)), ((), ())),
                        preferred_element_type=jnp.float32) + b1_ref[...]
    h = h * jax.nn.sigmoid(h)
    o_ref[...] = jnp.dot(h, w2_ref[...],
                         preferred_element_type=jnp.float32) + b2_ref[...]


def _post_body(a0_ref, a1_ref, x_ref, wr_ref, wg1_ref, bg1_ref,
               wg2_ref, bg2_ref, gam_ref, bet_ref, o_ref):
    agg = jnp.dot(a0_ref[0] + a1_ref[0], wr_ref[...],
                  preferred_element_type=jnp.float32)
    g = jnp.dot(agg, wg1_ref[...],
                preferred_element_type=jnp.float32) + bg1_ref[...]
    g = g * jax.nn.sigmoid(g)
    g = jnp.dot(g, wg2_ref[...],
                preferred_element_type=jnp.float32) + bg2_ref[...]
    y = x_ref[...] + g
    mu = jnp.mean(y, axis=-1, keepdims=True)
    yc = y - mu
    var = jnp.mean(yc * yc, axis=-1, keepdims=True)
    o_ref[...] = yc * jax.lax.rsqrt(var + 1e-5) * gam_ref[...] + bet_ref[...]


def _make_sc_edge_kernel(n_nodes, d, cpw, n_edges):
    zr = _CH
    rows_per_sub = -(-(-(-n_nodes // _NS)) // zr) * zr
    n_pad = rows_per_sub * _NS
    nzcopy = rows_per_sub // zr
    ncol = d // _LANES
    nbat = cpw // _BAT
    nchunks_real = n_edges // _CH

    def body(xs_hbm, phi_hbm, eidx_hbm, env_hbm, out_hbm,
             rows, phib, buf, bufe, acc, semA0, semA1, semS0, semS1, semB):
        c = lax.axis_index("c")
        s = lax.axis_index("s")
        wid = s * _NC + c
        semA = (semA0, semA1)
        semS = (semS0, semS1)

        def zfill(r, _):
            for cc in range(ncol):
                phib[0, r, pl.ds(cc * _LANES, _LANES)] = jnp.zeros(
                    (_LANES,), jnp.float32)
            return 0
        lax.fori_loop(0, zr, zfill, 0)

        def zcopy(j, _):
            pltpu.sync_copy(phib.at[0],
                            acc.at[pl.ds(s * rows_per_sub + j * zr, zr)])
            return 0
        lax.fori_loop(0, nzcopy, zcopy, 0)
        plsc.subcore_barrier()

        def fetch_batch(g, p, valid=True):
            row0 = wid * cpw + g * _BAT

            @pl.when(valid & (row0 < nchunks_real))
            def _():
                pltpu.async_copy(
                    eidx_hbm.at[pl.ds(0, 2), pl.ds(row0, _BAT)],
                    buf.at[p], semB)
                pltpu.async_copy(
                    env_hbm.at[pl.ds(row0, _BAT)], bufe.at[p], semB)

        def drain_batch(p):
            pltpu.make_async_copy(
                eidx_hbm.at[pl.ds(0, 2), pl.ds(0, _BAT)], buf.at[p],
                semB).wait()
            pltpu.make_async_copy(
                env_hbm.at[pl.ds(0, _BAT)], bufe.at[p], semB).wait()

        def fetch_chunk(g, p, j, b, may_drain_scatter, batch_boundary,
                        valid=True):
            base = (wid * cpw + g * _BAT + j) * _CH

            @pl.when(valid & (base < n_edges))
            def _():
                if batch_boundary:
                    drain_batch(p)
                if may_drain_scatter is True:
                    pltpu.make_async_copy(
                        phi_hbm.at[pl.ds(0, _CH)], rows.at[b],
                        semS[b]).wait()
                elif may_drain_scatter is not False:
                    @pl.when(may_drain_scatter)
                    def _():
                        pltpu.make_async_copy(
                            phi_hbm.at[pl.ds(0, _CH)], rows.at[b],
                            semS[b]).wait()
                pltpu.async_copy(xs_hbm.at[buf.at[p, 0, j]], rows.at[b],
                                 semA[b])
                pltpu.async_copy(phi_hbm.at[pl.ds(base, _CH)], phib.at[b],
                                 semA[b])

        def consume(g, p, j, b):
            base = (wid * cpw + g * _BAT + j) * _CH

            @pl.when(base < n_edges)
            def _():
                pltpu.make_async_copy(
                    phi_hbm.at[pl.ds(0, _CH)], rows.at[b], semA[b]).wait()
                pltpu.make_async_copy(
                    phi_hbm.at[pl.ds(0, _CH)], phib.at[b], semA[b]).wait()

                def mul_grp(jj, _):
                    ev = bufe[p, j, pl.ds(jj * _LANES, _LANES)]
                    for t in range(_LANES):
                        r = jj * _LANES + t
                        es = lax.gather(
                            ev,
                            jnp.full((_LANES, 1), t, jnp.int32),
                            lax.GatherDimensionNumbers(
                                offset_dims=(),
                                collapsed_slice_dims=(0,),
                                start_index_map=(0,)),
                            (1,),
                            mode=lax.GatherScatterMode.PROMISE_IN_BOUNDS)
                        for cc in range(ncol):
                            o = cc * _LANES
                            rows[b, r, pl.ds(o, _LANES)] = (
                                rows[b, r, pl.ds(o, _LANES)]
                                * (phib[b, r, pl.ds(o, _LANES)] * es))
                    return 0
                lax.fori_loop(0, _CH // _LANES, mul_grp, 0)
                pltpu.async_copy(rows.at[b], acc.at[buf.at[p, 1, j]],
                                 semS[b], add=True)

        fetch_batch(0, 0)
        drain_batch(0)
        fetch_chunk(0, 0, 0, 0, False, False)

        def super_body(gg, _):
            for pp in (0, 1):
                g = 2 * gg + pp
                nxt = g < (nbat - 1)
                fetch_batch(g + 1, 1 - pp, valid=nxt)
                for j in range(_BAT):
                    b = j & 1
                    if j + 1 < _BAT:
                        may = True if j + 1 >= 2 else (g > 0)
                        fetch_chunk(g, pp, j + 1, b ^ 1, may, False)
                    else:
                        fetch_chunk(g + 1, 1 - pp, 0, b ^ 1, True, True,
                                    valid=nxt)
                    consume(g, pp, j, b)
            return 0
        lax.fori_loop(0, nbat // 2, super_body, 0)

        kr = jnp.clip(nchunks_real - wid * cpw, 0, cpw)
        for b in (0, 1):
            pred = (kr >= 2) | ((kr == 1) & (b == 0))

            @pl.when(pred)
            def _():
                pltpu.make_async_copy(
                    phi_hbm.at[pl.ds(0, _CH)], rows.at[b], semS[b]).wait()
        plsc.subcore_barrier()

        pltpu.sync_copy(
            acc.at[pl.ds(s * rows_per_sub, rows_per_sub)],
            out_hbm.at[c, pl.ds(s * rows_per_sub, rows_per_sub)])

    mesh = plsc.VectorSubcoreMesh(core_axis_name="c", subcore_axis_name="s")
    return pl.kernel(
        body,
        mesh=mesh,
        out_type=jax.ShapeDtypeStruct((_NC, n_pad, d), jnp.float32),
        scratch_types=[
            pltpu.VMEM((2, _CH, d), jnp.float32),
            pltpu.VMEM((2, _CH, d), jnp.float32),
            pltpu.VMEM((2, 2, _BAT, _CH), jnp.int32),
            pltpu.VMEM((2, _BAT, _CH), jnp.float32),
            pltpu.VMEM_SHARED((n_pad, d), jnp.float32),
            pltpu.SemaphoreType.DMA,
            pltpu.SemaphoreType.DMA,
            pltpu.SemaphoreType.DMA,
            pltpu.SemaphoreType.DMA,
            pltpu.SemaphoreType.DMA,
        ],
    )


def kernel(x, edge_index, rbf, envelope, W_send, W_recv,
           Wf1, bf1, Wf2, bf2, Wg1, bg1, Wg2, bg2, gamma, beta):
    N, d = x.shape
    E = edge_index.shape[1]
    nr = rbf.shape[1]
    ds = W_send.shape[0]

    assert E % _CH == 0
    step = 2 * _BAT
    cpw = -(-E // (_NW * _CH))
    cpw = -(-cpw // step) * step
    eidx = edge_index.astype(jnp.int32).reshape(2, E // _CH, _CH)
    envr = envelope.reshape(E // _CH, _CH)

    gN = N // _BN
    xs = pl.pallas_call(
        _xs_body,
        grid=(gN,),
        in_specs=[pl.BlockSpec((_BN, d), lambda i: (i, 0)),
                  _full_spec((d, ds))],
        out_specs=pl.BlockSpec((_BN, ds), lambda i: (i, 0)),
        out_shape=jax.ShapeDtypeStruct((N, ds), jnp.float32),
        compiler_params=pltpu.CompilerParams(
            dimension_semantics=("parallel",)),
    )(x, W_send.T)

    be = next(b for b in (6400, 3200, 2560, 2048, 1280, 1024, 640, 512, 128)
              if E % b == 0)
    gE = E // be
    phi = pl.pallas_call(
        _phi_body,
        grid=(gE,),
        in_specs=[pl.BlockSpec((nr, be), lambda i: (0, i)),
                  _full_spec((nr, d)),
                  _full_spec((1, d)),
                  _full_spec((d, ds)),
                  _full_spec((1, ds))],
        out_specs=pl.BlockSpec((be, ds), lambda i: (i, 0)),
        out_shape=jax.ShapeDtypeStruct((E, ds), jnp.float32),
        compiler_params=pltpu.CompilerParams(
            dimension_semantics=("parallel",)),
    )(rbf.T, Wf1.T, bf1[None, :], Wf2.T, bf2[None, :])

    acc2 = _make_sc_edge_kernel(N, ds, cpw, E)(xs, phi, eidx, envr)

    out = pl.pallas_call(
        _post_body,
        grid=(gN,),
        in_specs=[pl.BlockSpec((1, _BN, ds), lambda i: (0, i, 0)),
                  pl.BlockSpec((1, _BN, ds), lambda i: (1, i, 0)),
                  pl.BlockSpec((_BN, d), lambda i: (i, 0)),
                  _full_spec((ds, d)),
                  _full_spec((d, d)),
                  _full_spec((1, d)),
                  _full_spec((d, d)),
                  _full_spec((1, d)),
                  _full_spec((1, d)),
                  _full_spec((1, d))],
        out_specs=pl.BlockSpec((_BN, d), lambda i: (i, 0)),
        out_shape=jax.ShapeDtypeStruct((N, d), jnp.float32),
        compiler_params=pltpu.CompilerParams(
            dimension_semantics=("parallel",)),
    )(acc2, acc2, x, W_recv, Wg1.T, bg1[None, :], Wg2.T, bg2[None, :],
      gamma[None, :], beta[None, :])
    return out


def _selfcheck_note():
    pass


# speedup vs baseline: 1.1342x; 1.1342x over previous
import functools

import jax
import jax.numpy as jnp
from jax import lax
from jax.experimental import pallas as pl
from jax.experimental.pallas import tpu as pltpu
from jax.experimental.pallas import tpu_sc as plsc

_NC = 2
_NS = 16
_NW = _NC * _NS
_LANES = 16
_CH = 64
_BAT = 4
_BN = 1000
_BE = 2048


def _full_spec(shape):
    nd = len(shape)
    return pl.BlockSpec(shape, lambda i, _n=nd: (0,) * _n)


def _xs_body(x_ref, w_ref, o_ref):
    o_ref[...] = jnp.dot(x_ref[...], w_ref[...],
                         preferred_element_type=jnp.float32)


def _phi_body(rbf_t_ref, w1_ref, b1_ref, w2_ref, b2_ref, o_ref):
    h = lax.dot_general(rbf_t_ref[...], w1_ref[...],
                        (((0,), (0,)), ((), ())),
                        preferred_element_type=jnp.float32) + b1_ref[...]
    h = h * jax.nn.sigmoid(h)
    phi = jnp.dot(h, w2_ref[...],
                  preferred_element_type=jnp.float32) + b2_ref[...]
    half = phi.shape[1] // 2
    lo = lax.bitcast_convert_type(phi[:, :half], jnp.int32)
    hi = lax.bitcast_convert_type(phi[:, half:], jnp.int32)
    wlo = lax.shift_right_logical(lo + 32768, 16)
    whi = (hi + 32768) & jnp.int32(-65536)
    o_ref[...] = whi | wlo


def _post_body(a0_ref, a1_ref, x_ref, wr_ref, wg1_ref, bg1_ref,
               wg2_ref, bg2_ref, gam_ref, bet_ref, o_ref):
    agg = jnp.dot(a0_ref[0] + a1_ref[0], wr_ref[...],
                  preferred_element_type=jnp.float32)
    g = jnp.dot(agg, wg1_ref[...],
                preferred_element_type=jnp.float32) + bg1_ref[...]
    g = g * jax.nn.sigmoid(g)
    g = jnp.dot(g, wg2_ref[...],
                preferred_element_type=jnp.float32) + bg2_ref[...]
    y = x_ref[...] + g
    mu = jnp.mean(y, axis=-1, keepdims=True)
    yc = y - mu
    var = jnp.mean(yc * yc, axis=-1, keepdims=True)
    o_ref[...] = yc * jax.lax.rsqrt(var + 1e-5) * gam_ref[...] + bet_ref[...]


def _make_sc_edge_kernel(n_nodes, d, cpw, n_edges):
    zr = _CH
    rows_per_sub = -(-(-(-n_nodes // _NS)) // zr) * zr
    n_pad = rows_per_sub * _NS
    nzcopy = rows_per_sub // zr
    ncol = d // _LANES
    nbat = cpw // _BAT
    nchunks_real = n_edges // _CH

    def body(xs_hbm, phi_hbm, eidx_hbm, env_hbm, out_hbm,
             rows, phib, buf, bufe, acc, semA0, semA1, semS0, semS1, semB):
        c = lax.axis_index("c")
        s = lax.axis_index("s")
        wid = s * _NC + c
        semA = (semA0, semA1)
        semS = (semS0, semS1)

        def zfill(r, _):
            for cc in range(ncol):
                rows[0, r, pl.ds(cc * _LANES, _LANES)] = jnp.zeros(
                    (_LANES,), jnp.float32)
            return 0
        lax.fori_loop(0, zr, zfill, 0)

        def zcopy(j, _):
            pltpu.sync_copy(rows.at[0],
                            acc.at[pl.ds(s * rows_per_sub + j * zr, zr)])
            return 0
        lax.fori_loop(0, nzcopy, zcopy, 0)
        plsc.subcore_barrier()

        def fetch_batch(g, p, valid=True):
            row0 = wid * cpw + g * _BAT

            @pl.when(valid & (row0 < nchunks_real))
            def _():
                pltpu.async_copy(
                    eidx_hbm.at[pl.ds(0, 2), pl.ds(row0, _BAT)],
                    buf.at[p], semB)
                pltpu.async_copy(
                    env_hbm.at[pl.ds(row0, _BAT)], bufe.at[p], semB)

        def drain_batch(p):
            pltpu.make_async_copy(
                eidx_hbm.at[pl.ds(0, 2), pl.ds(0, _BAT)], buf.at[p],
                semB).wait()
            pltpu.make_async_copy(
                env_hbm.at[pl.ds(0, _BAT)], bufe.at[p], semB).wait()

        def fetch_chunk(g, p, j, b, may_drain_scatter, batch_boundary,
                        valid=True):
            base = (wid * cpw + g * _BAT + j) * _CH

            @pl.when(valid & (base < n_edges))
            def _():
                if batch_boundary:
                    drain_batch(p)
                if may_drain_scatter is True:
                    pltpu.make_async_copy(
                        xs_hbm.at[pl.ds(0, _CH)], rows.at[b],
                        semS[b]).wait()
                elif may_drain_scatter is not False:
                    @pl.when(may_drain_scatter)
                    def _():
                        pltpu.make_async_copy(
                            xs_hbm.at[pl.ds(0, _CH)], rows.at[b],
                            semS[b]).wait()
                pltpu.async_copy(xs_hbm.at[buf.at[p, 0, j]], rows.at[b],
                                 semA[b])
                pltpu.async_copy(phi_hbm.at[pl.ds(base, _CH)], phib.at[b],
                                 semA[b])

        def consume(g, p, j, b):
            base = (wid * cpw + g * _BAT + j) * _CH

            @pl.when(base < n_edges)
            def _():
                pltpu.make_async_copy(
                    xs_hbm.at[pl.ds(0, _CH)], rows.at[b], semA[b]).wait()
                pltpu.make_async_copy(
                    phi_hbm.at[pl.ds(0, _CH)], phib.at[b], semA[b]).wait()

                def mul_grp(jj, _):
                    ev = bufe[p, j, pl.ds(jj * _LANES, _LANES)]
                    for t in range(_LANES):
                        r = jj * _LANES + t
                        es = lax.gather(
                            ev,
                            jnp.full((_LANES, 1), t, jnp.int32),
                            lax.GatherDimensionNumbers(
                                offset_dims=(),
                                collapsed_slice_dims=(0,),
                                start_index_map=(0,)),
                            (1,),
                            mode=lax.GatherScatterMode.PROMISE_IN_BOUNDS)
                        for cc in range(ncol // 2):
                            o = cc * 2 * _LANES
                            w = phib[b, r, pl.ds(cc * _LANES, _LANES)]
                            pa = lax.bitcast_convert_type(
                                w << 16, jnp.float32)
                            pb = lax.bitcast_convert_type(
                                w & jnp.int32(-65536), jnp.float32)
                            rows[b, r, pl.ds(o, _LANES)] = (
                                rows[b, r, pl.ds(o, _LANES)] * (pa * es))
                            rows[b, r, pl.ds(o + _LANES, _LANES)] = (
                                rows[b, r, pl.ds(o + _LANES, _LANES)]
                                * (pb * es))
                    return 0
                lax.fori_loop(0, _CH // _LANES, mul_grp, 0)
                pltpu.async_copy(rows.at[b], acc.at[buf.at[p, 1, j]],
                                 semS[b], add=True)

        fetch_batch(0, 0)
        drain_batch(0)
        fetch_chunk(0, 0, 0, 0, False, False)

        def super_body(gg, _):
            for pp in (0, 1):
                g = 2 * gg + pp
                nxt = g < (nbat - 1)
                fetch_batch(g + 1, 1 - pp, valid=nxt)
                for j in range(_BAT):
                    b = j & 1
                    if j + 1 < _BAT:
                        may = True if j + 1 >= 2 else (g > 0)
                        fetch_chunk(g, pp, j + 1, b ^ 1, may, False)
                    else:
                        fetch_chunk(g + 1, 1 - pp, 0, b ^ 1, True, True,
                                    valid=nxt)
                    consume(g, pp, j, b)
            return 0
        lax.fori_loop(0, nbat // 2, super_body, 0)

        kr = jnp.clip(nchunks_real - wid * cpw, 0, cpw)
        for b in (0, 1):
            pred = (kr >= 2) | ((kr == 1) & (b == 0))

            @pl.when(pred)
            def _():
                pltpu.make_async_copy(
                    xs_hbm.at[pl.ds(0, _CH)], rows.at[b], semS[b]).wait()
        plsc.subcore_barrier()

        pltpu.sync_copy(
            acc.at[pl.ds(s * rows_per_sub, rows_per_sub)],
            out_hbm.at[c, pl.ds(s * rows_per_sub, rows_per_sub)])

    mesh = plsc.VectorSubcoreMesh(core_axis_name="c", subcore_axis_name="s")
    return pl.kernel(
        body,
        mesh=mesh,
        out_type=jax.ShapeDtypeStruct((_NC, n_pad, d), jnp.float32),
        scratch_types=[
            pltpu.VMEM((2, _CH, d), jnp.float32),
            pltpu.VMEM((2, _CH, d // 2), jnp.int32),
            pltpu.VMEM((2, 2, _BAT, _CH), jnp.int32),
            pltpu.VMEM((2, _BAT, _CH), jnp.float32),
            pltpu.VMEM_SHARED((n_pad, d), jnp.float32),
            pltpu.SemaphoreType.DMA,
            pltpu.SemaphoreType.DMA,
            pltpu.SemaphoreType.DMA,
            pltpu.SemaphoreType.DMA,
            pltpu.SemaphoreType.DMA,
        ],
    )


def kernel(x, edge_index, rbf, envelope, W_send, W_recv,
           Wf1, bf1, Wf2, bf2, Wg1, bg1, Wg2, bg2, gamma, beta):
    N, d = x.shape
    E = edge_index.shape[1]
    nr = rbf.shape[1]
    ds = W_send.shape[0]

    assert E % _CH == 0
    step = 2 * _BAT
    cpw = -(-E // (_NW * _CH))
    cpw = -(-cpw // step) * step
    eidx = edge_index.astype(jnp.int32).reshape(2, E // _CH, _CH)
    envr = envelope.reshape(E // _CH, _CH)

    half = ds // 2
    perm = jnp.array([c
                      for cc in range(half // 16)
                      for c in (list(range(16 * cc, 16 * cc + 16))
                                + list(range(half + 16 * cc,
                                             half + 16 * cc + 16)))],
                     dtype=jnp.int32)

    gN = N // _BN
    xs = pl.pallas_call(
        _xs_body,
        grid=(gN,),
        in_specs=[pl.BlockSpec((_BN, d), lambda i: (i, 0)),
                  _full_spec((d, ds))],
        out_specs=pl.BlockSpec((_BN, ds), lambda i: (i, 0)),
        out_shape=jax.ShapeDtypeStruct((N, ds), jnp.float32),
        compiler_params=pltpu.CompilerParams(
            dimension_semantics=("parallel",)),
    )(x, W_send[perm].T)

    be = next(b for b in (6400, 3200, 2560, 2048, 1280, 1024, 640, 512, 128)
              if E % b == 0)
    gE = E // be
    phi = pl.pallas_call(
        _phi_body,
        grid=(gE,),
        in_specs=[pl.BlockSpec((nr, be), lambda i: (0, i)),
                  _full_spec((nr, d)),
                  _full_spec((1, d)),
                  _full_spec((d, ds)),
                  _full_spec((1, ds))],
        out_specs=pl.BlockSpec((be, ds // 2), lambda i: (i, 0)),
        out_shape=jax.ShapeDtypeStruct((E, ds // 2), jnp.int32),
        compiler_params=pltpu.CompilerParams(
            dimension_semantics=("parallel",)),
    )(rbf.T, Wf1.T, bf1[None, :], Wf2.T, bf2[None, :])

    acc2 = _make_sc_edge_kernel(N, ds, cpw, E)(xs, phi, eidx, envr)

    out = pl.pallas_call(
        _post_body,
        grid=(gN,),
        in_specs=[pl.BlockSpec((1, _BN, ds), lambda i: (0, i, 0)),
                  pl.BlockSpec((1, _BN, ds), lambda i: (1, i, 0)),
                  pl.BlockSpec((_BN, d), lambda i: (i, 0)),
                  _full_spec((ds, d)),
                  _full_spec((d, d)),
                  _full_spec((1, d)),
                  _full_spec((d, d)),
                  _full_spec((1, d)),
                  _full_spec((1, d)),
                  _full_spec((1, d))],
        out_specs=pl.BlockSpec((_BN, d), lambda i: (i, 0)),
        out_shape=jax.ShapeDtypeStruct((N, d), jnp.float32),
        compiler_params=pltpu.CompilerParams(
            dimension_semantics=("parallel",)),
    )(acc2, acc2, x, W_recv[perm], Wg1.T, bg1[None, :], Wg2.T, bg2[None, :],
      gamma[None, :], beta[None, :])
    return out


def _selfcheck_note():
    pass
